# Initial kernel scaffold; baseline (speedup 1.0000x reference)
import jax
import jax.numpy as jnp
from jax.experimental import pallas as pl


def kernel(node_features, edge_index, line_flows, thermal_limits, susceptance):
    raise NotImplementedError("write your pallas kernel here")


# speedup vs baseline: 10.3569x; 10.3569x over previous
import functools

import jax
import jax.numpy as jnp
from jax import lax
from jax.experimental import pallas as pl
from jax.experimental.pallas import tpu as pltpu
from jax.experimental.pallas import tpu_sc as plsc

B, N, F, E = 4, 10000, 128, 320000

NC, NS, LANES = 2, 16, 16
NTILES = NC * NS
EDGES_PER_TILE = E // NTILES
K = 400
SUB = 5
KSUB = K // SUB
GROUPS = K // LANES
CHUNKS = EDGES_PER_TILE // K
ACC_W = 16
NBLK = N // 8
WSUB = 10
BLKS = NBLK // WSUB


@functools.cache
def _make_sc_edge_scatter():
    sc_mesh = plsc.VectorSubcoreMesh(
        core_axis_name="c", subcore_axis_name="s", num_cores=NC, num_subcores=NS
    )
    return pl.kernel(
        _sc_edge_scatter_body,
        out_type=jax.ShapeDtypeStruct((NC, N, ACC_W), jnp.float32),
        mesh=sc_mesh,
        compiler_params=pltpu.CompilerParams(
            use_tc_tiling_on_sc=False, needs_layout_passes=False),
        scratch_types=[
            pltpu.VMEM((B * N,), jnp.float32),
            pltpu.VMEM((SUB, KSUB), jnp.int32),
            pltpu.VMEM((SUB, KSUB), jnp.int32),
            pltpu.VMEM((B, K), jnp.float32),
            pltpu.VMEM((B, K), jnp.float32),
            pltpu.VMEM((B, K), jnp.float32),
            pltpu.VMEM((SUB, KSUB, ACC_W), jnp.float32),
            pltpu.VMEM((8, ACC_W), jnp.float32),
            pltpu.VMEM_SHARED((N, ACC_W), jnp.float32),
            pltpu.SemaphoreType.DMA,
            pltpu.SemaphoreType.DMA,
        ],
    )


def _sc_edge_scatter_body(volt_hbm, src_hbm, dst_hbm, lf_hbm, tl_hbm, sus_hbm,
                          acc_hbm, volt_v, src_v, dst_v, lf_v, tl_v, sus_v,
                          rows_v, zbuf, acc_sh, sem_in, sem_v):
    core = lax.axis_index("c")
    sub = lax.axis_index("s")

    cp_volt = pltpu.async_copy(volt_hbm, volt_v, sem_v)

    z16 = jnp.zeros((LANES,), jnp.float32)

    @pl.loop(0, 8)
    def _(r):
        zbuf[r, :] = z16

    @pl.loop(0, SUB)
    def _(j):
        @pl.loop(0, KSUB)
        def _(k):
            rows_v[j, k, :] = z16

    @pl.when(sub < WSUB)
    def _():
        @pl.loop(0, BLKS)
        def _(i):
            blk = sub * BLKS + i
            pltpu.sync_copy(zbuf, acc_sh.at[pl.ds(blk * 8, 8)])

    plsc.subcore_barrier()
    cp_volt.wait()

    ebase = (core * NS + sub) * EDGES_PER_TILE
    iota = lax.iota(jnp.int32, LANES)

    @pl.loop(0, CHUNKS)
    def _(c):
        e0 = ebase + c * K
        cps = []
        for j in range(SUB):
            cps.append(pltpu.async_copy(
                src_hbm.at[pl.ds(e0 + j * KSUB, KSUB)], src_v.at[j], sem_in))
            cps.append(pltpu.async_copy(
                dst_hbm.at[pl.ds(e0 + j * KSUB, KSUB)], dst_v.at[j], sem_in))
        for b in range(B):
            cps.append(pltpu.async_copy(
                lf_hbm.at[pl.ds(b * E + e0, K)], lf_v.at[b], sem_in))
            cps.append(pltpu.async_copy(
                tl_hbm.at[pl.ds(b * E + e0, K)], tl_v.at[b], sem_in))
            cps.append(pltpu.async_copy(
                sus_hbm.at[pl.ds(b * E + e0, K)], sus_v.at[b], sem_in))
        for cp in cps:
            cp.wait()

        for g in range(GROUPS):
            j, m = divmod(g, SUB)
            s16 = src_v[j, pl.ds(m * LANES, LANES)]
            d16 = dst_v[j, pl.ds(m * LANES, LANES)]
            kidx = iota + (m * LANES)
            jsplat = jnp.full((LANES,), j, jnp.int32)
            for b in range(B):
                sb = s16 + (b * N) if b else s16
                db = d16 + (b * N) if b else d16
                vi = plsc.load_gather(volt_v, [sb])
                vj = plsc.load_gather(volt_v, [db])
                sus16 = sus_v[b, pl.ds(g * LANES, LANES)]
                fij = sus16 * vj / (vi + 1e-6)
                plsc.store_scatter(
                    rows_v, [jsplat, kidx, jnp.full((LANES,), b, jnp.int32)], fij)
                lf16 = lf_v[b, pl.ds(g * LANES, LANES)]
                tl16 = tl_v[b, pl.ds(g * LANES, LANES)]
                loading = lf16 / (tl16 + 1e-6)
                ind = jnp.where(loading > 1.0, 1.0, 0.0).astype(jnp.float32)
                plsc.store_scatter(
                    rows_v, [jsplat, kidx, jnp.full((LANES,), 4 + b, jnp.int32)], ind)

        for j in range(SUB):
            pltpu.sync_copy(rows_v.at[j], acc_sh.at[src_v.at[j]], add=True)

    plsc.subcore_barrier()

    @pl.when(sub < WSUB)
    def _():
        @pl.loop(0, BLKS)
        def _(i):
            blk = sub * BLKS + i
            pltpu.sync_copy(acc_sh.at[pl.ds(blk * 8, 8)],
                            acc_hbm.at[core, pl.ds(blk * 8, 8), :])


BN = 2000


def _tc_body(nf_ref, acc_ref, out_ref):
    b = pl.program_id(0)
    nf = nf_ref[0]
    a = acc_ref[0] + acc_ref[1]
    lane = lax.broadcasted_iota(jnp.int32, (BN, ACC_W), 1)
    fsum = jnp.sum(jnp.where(lane == b, a, 0.0), axis=1)
    gcnt = jnp.sum(jnp.where(lane == b + 4, a, 0.0), axis=1)
    n1 = (gcnt > 0.0).astype(jnp.float32)
    li = jnp.abs(1.0 - fsum)
    out_ref[0, :, 0:128] = nf
    out_ref[0, :, 128] = n1
    out_ref[0, :, 129] = li


_tc_finalize = pl.pallas_call(
    _tc_body,
    grid=(B, N // BN),
    in_specs=[
        pl.BlockSpec((1, BN, F), lambda b, i: (b, i, 0)),
        pl.BlockSpec((NC, BN, ACC_W), lambda b, i: (0, i, 0)),
    ],
    out_specs=pl.BlockSpec((1, BN, F + 2), lambda b, i: (b, i, 0)),
    out_shape=jax.ShapeDtypeStruct((B, N, F + 2), jnp.float32),
)


def kernel(node_features, edge_index, line_flows, thermal_limits, susceptance):
    volt = node_features[:, :, 0].reshape(B * N)
    src = edge_index[0]
    dst = edge_index[1]
    lf1 = line_flows[:, :, 0].reshape(B * E)
    tl1 = thermal_limits[:, :, 0].reshape(B * E)
    sus1 = susceptance[:, :, 0].reshape(B * E)
    acc = _make_sc_edge_scatter()(volt, src, dst, lf1, tl1, sus1)
    return _tc_finalize(node_features, acc)


# speedup vs baseline: 12.6512x; 1.2215x over previous
import functools

import jax
import jax.numpy as jnp
from jax import lax
from jax.experimental import pallas as pl
from jax.experimental.pallas import tpu as pltpu
from jax.experimental.pallas import tpu_sc as plsc

B, N, F, E = 4, 10000, 128, 320000

NC, NS, LANES = 2, 16, 16
NTILES = NC * NS
EDGES_PER_TILE = E // NTILES
K = 400
GROUPS = K // LANES
CHUNKS = EDGES_PER_TILE // K
ACC_W = 8
NPAD = 10240
ACC_LEN = ACC_W * NPAD


@functools.cache
def _make_sc_edge_scatter():
    sc_mesh = plsc.VectorSubcoreMesh(
        core_axis_name="c", subcore_axis_name="s", num_cores=NC, num_subcores=NS
    )
    return pl.kernel(
        _sc_edge_scatter_body,
        out_type=jax.ShapeDtypeStruct((NTILES * ACC_LEN,), jnp.float32),
        mesh=sc_mesh,
        compiler_params=pltpu.CompilerParams(
            use_tc_tiling_on_sc=False, needs_layout_passes=False),
        scratch_types=[
            pltpu.VMEM((B * N,), jnp.float32),
            pltpu.VMEM((ACC_LEN,), jnp.float32),
            pltpu.VMEM((K,), jnp.int32),
            pltpu.VMEM((K,), jnp.int32),
            pltpu.VMEM((B, K), jnp.float32),
            pltpu.VMEM((B, K), jnp.float32),
            pltpu.VMEM((B, K), jnp.float32),
            pltpu.SemaphoreType.DMA,
            pltpu.SemaphoreType.DMA,
        ],
    )


def _sc_edge_scatter_body(volt_hbm, src_hbm, dst_hbm, lf_hbm, tl_hbm, sus_hbm,
                          acc_hbm, volt_v, acc_v, src_v, dst_v, lf_v, tl_v,
                          sus_v, sem_in, sem_v):
    core = lax.axis_index("c")
    sub = lax.axis_index("s")

    cp_volt = pltpu.async_copy(volt_hbm, volt_v, sem_v)

    z16 = jnp.zeros((LANES,), jnp.float32)

    @pl.loop(0, ACC_LEN // LANES)
    def _(i):
        acc_v[pl.ds(i * LANES, LANES)] = z16

    cp_volt.wait()

    ebase = (core * NS + sub) * EDGES_PER_TILE

    @pl.loop(0, CHUNKS)
    def _(c):
        e0 = ebase + c * K
        cps = [
            pltpu.async_copy(src_hbm.at[pl.ds(e0, K)], src_v, sem_in),
            pltpu.async_copy(dst_hbm.at[pl.ds(e0, K)], dst_v, sem_in),
        ]
        for b in range(B):
            cps.append(pltpu.async_copy(
                lf_hbm.at[pl.ds(b * E + e0, K)], lf_v.at[b], sem_in))
            cps.append(pltpu.async_copy(
                tl_hbm.at[pl.ds(b * E + e0, K)], tl_v.at[b], sem_in))
            cps.append(pltpu.async_copy(
                sus_hbm.at[pl.ds(b * E + e0, K)], sus_v.at[b], sem_in))
        for cp in cps:
            cp.wait()

        for g in range(GROUPS):
            s16 = src_v[pl.ds(g * LANES, LANES)]
            d16 = dst_v[pl.ds(g * LANES, LANES)]
            for b in range(B):
                sb = s16 + (b * N) if b else s16
                db = d16 + (b * N) if b else d16
                vi = plsc.load_gather(volt_v, [sb])
                vj = plsc.load_gather(volt_v, [db])
                sus16 = sus_v[b, pl.ds(g * LANES, LANES)]
                fij = sus16 * vj / (vi + 1e-6)
                sa = s16 + (b * NPAD) if b else s16
                plsc.addupdate_scatter(acc_v, [sa], fij)
                lf16 = lf_v[b, pl.ds(g * LANES, LANES)]
                tl16 = tl_v[b, pl.ds(g * LANES, LANES)]
                loading = lf16 / (tl16 + 1e-6)
                ind = jnp.where(loading > 1.0, 1.0, 0.0).astype(jnp.float32)
                plsc.addupdate_scatter(acc_v, [sa + (4 * NPAD)], ind)

    tile = core * NS + sub
    pltpu.sync_copy(acc_v, acc_hbm.at[pl.ds(tile * ACC_LEN, ACC_LEN)])


BN = 2048


def _tc_body(nf_ref, acc_ref, out_ref):
    b = pl.program_id(0)
    nf = nf_ref[0]
    a = acc_ref[0, 0]
    for t in range(1, NTILES):
        a = a + acc_ref[0, t]
    row = lax.broadcasted_iota(jnp.int32, (ACC_W, BN), 0)
    fsum = jnp.sum(jnp.where(row == b, a, 0.0), axis=0)
    gcnt = jnp.sum(jnp.where(row == b + 4, a, 0.0), axis=0)
    n1 = (gcnt > 0.0).astype(jnp.float32)
    li = jnp.abs(1.0 - fsum)
    out_ref[0, :, 0:128] = nf
    out_ref[0, :, 128] = n1
    out_ref[0, :, 129] = li


_tc_finalize = pl.pallas_call(
    _tc_body,
    grid=(B, NPAD // BN),
    in_specs=[
        pl.BlockSpec((1, BN, F), lambda b, i: (b, i, 0)),
        pl.BlockSpec((1, NTILES, ACC_W, BN), lambda b, i: (0, 0, 0, i)),
    ],
    out_specs=pl.BlockSpec((1, BN, F + 2), lambda b, i: (b, i, 0)),
    out_shape=jax.ShapeDtypeStruct((B, N, F + 2), jnp.float32),
)


def kernel(node_features, edge_index, line_flows, thermal_limits, susceptance):
    volt = node_features[:, :, 0].reshape(B * N)
    src = edge_index[0]
    dst = edge_index[1]
    lf1 = line_flows[:, :, 0].reshape(B * E)
    tl1 = thermal_limits[:, :, 0].reshape(B * E)
    sus1 = susceptance[:, :, 0].reshape(B * E)
    acc = _make_sc_edge_scatter()(volt, src, dst, lf1, tl1, sus1)
    return _tc_finalize(node_features, acc.reshape(1, NTILES, ACC_W, NPAD))


# speedup vs baseline: 13.2587x; 1.0480x over previous
import functools

import jax
import jax.numpy as jnp
from jax import lax
from jax.experimental import pallas as pl
from jax.experimental.pallas import tpu as pltpu
from jax.experimental.pallas import tpu_sc as plsc

B, N, F, E = 4, 10000, 128, 320000

NC, NS, LANES = 2, 16, 16
NTILES = NC * NS
EDGES_PER_TILE = E // NTILES
K = 400
GROUPS = K // LANES
CHUNKS = EDGES_PER_TILE // K
ACC_W = 8
NPAD = 10240
ACC_LEN = ACC_W * NPAD


@functools.cache
def _make_sc_edge_scatter():
    sc_mesh = plsc.VectorSubcoreMesh(
        core_axis_name="c", subcore_axis_name="s", num_cores=NC, num_subcores=NS
    )
    return pl.kernel(
        _sc_edge_scatter_body,
        out_type=jax.ShapeDtypeStruct((NTILES * ACC_LEN,), jnp.float32),
        mesh=sc_mesh,
        compiler_params=pltpu.CompilerParams(
            use_tc_tiling_on_sc=False, needs_layout_passes=False),
        scratch_types=[
            pltpu.VMEM((B * N,), jnp.float32),
            pltpu.VMEM((ACC_LEN,), jnp.float32),
            pltpu.VMEM((K,), jnp.int32),
            pltpu.VMEM((K,), jnp.int32),
            pltpu.VMEM((B, K), jnp.float32),
            pltpu.VMEM((B, K), jnp.float32),
            pltpu.VMEM((B, K), jnp.float32),
            pltpu.SemaphoreType.DMA,
            pltpu.SemaphoreType.DMA,
        ],
    )


def _sc_edge_scatter_body(volt_hbm, src_hbm, dst_hbm, lf_hbm, tl_hbm, sus_hbm,
                          acc_hbm, volt_v, acc_v, src_v, dst_v, lf_v, tl_v,
                          sus_v, sem_in, sem_v):
    core = lax.axis_index("c")
    sub = lax.axis_index("s")

    cp_volt = pltpu.async_copy(volt_hbm, volt_v, sem_v)

    z16 = jnp.zeros((LANES,), jnp.float32)

    @pl.loop(0, ACC_LEN // LANES)
    def _(i):
        acc_v[pl.ds(i * LANES, LANES)] = z16

    cp_volt.wait()

    ebase = (core * NS + sub) * EDGES_PER_TILE

    @pl.loop(0, CHUNKS)
    def _(c):
        e0 = ebase + c * K
        cps = [
            pltpu.async_copy(src_hbm.at[pl.ds(e0, K)], src_v, sem_in),
            pltpu.async_copy(dst_hbm.at[pl.ds(e0, K)], dst_v, sem_in),
        ]
        for b in range(B):
            cps.append(pltpu.async_copy(
                lf_hbm.at[pl.ds(b * E + e0, K)], lf_v.at[b], sem_in))
            cps.append(pltpu.async_copy(
                tl_hbm.at[pl.ds(b * E + e0, K)], tl_v.at[b], sem_in))
            cps.append(pltpu.async_copy(
                sus_hbm.at[pl.ds(b * E + e0, K)], sus_v.at[b], sem_in))
        for cp in cps:
            cp.wait()

        for g in range(GROUPS):
            s16 = src_v[pl.ds(g * LANES, LANES)]
            d16 = dst_v[pl.ds(g * LANES, LANES)]
            for b in range(B):
                sb = s16 + (b * N) if b else s16
                db = d16 + (b * N) if b else d16
                vi = plsc.load_gather(volt_v, [sb])
                vj = plsc.load_gather(volt_v, [db])
                sus16 = sus_v[b, pl.ds(g * LANES, LANES)]
                fij = sus16 * vj / (vi + 1e-6)
                sa = s16 + (b * NPAD) if b else s16
                plsc.addupdate_scatter(acc_v, [sa], fij)
                lf16 = lf_v[b, pl.ds(g * LANES, LANES)]
                tl16 = tl_v[b, pl.ds(g * LANES, LANES)]
                loading = lf16 / (tl16 + 1e-6)
                ind = jnp.where(loading > 1.0, 1.0, 0.0).astype(jnp.float32)
                plsc.addupdate_scatter(acc_v, [sa + (4 * NPAD)], ind)

    tile = core * NS + sub
    pltpu.sync_copy(acc_v, acc_hbm.at[pl.ds(tile * ACC_LEN, ACC_LEN)])


BN = 2048


def _tc_body(nf_ref, acc_ref, out_ref):
    a = acc_ref[0, 0]
    for t in range(1, NTILES):
        a = a + acc_ref[0, t]
    out_ref[:, :, 0:F] = nf_ref[...]
    for b in range(B):
        out_ref[b, :, F] = (a[B + b] > 0.0).astype(jnp.float32)
        out_ref[b, :, F + 1] = jnp.abs(1.0 - a[b])


_tc_finalize = pl.pallas_call(
    _tc_body,
    grid=(NPAD // BN,),
    in_specs=[
        pl.BlockSpec((B, BN, F), lambda i: (0, i, 0)),
        pl.BlockSpec((1, NTILES, ACC_W, BN), lambda i: (0, 0, 0, i)),
    ],
    out_specs=pl.BlockSpec((B, BN, F + 2), lambda i: (0, i, 0)),
    out_shape=jax.ShapeDtypeStruct((B, N, F + 2), jnp.float32),
)


def kernel(node_features, edge_index, line_flows, thermal_limits, susceptance):
    volt = node_features[:, :, 0].reshape(B * N)
    src = edge_index[0]
    dst = edge_index[1]
    lf1 = line_flows[:, :, 0].reshape(B * E)
    tl1 = thermal_limits[:, :, 0].reshape(B * E)
    sus1 = susceptance[:, :, 0].reshape(B * E)
    acc = _make_sc_edge_scatter()(volt, src, dst, lf1, tl1, sus1)
    return _tc_finalize(node_features, acc.reshape(1, NTILES, ACC_W, NPAD))
